# baseline (speedup 1.0000x reference)
import functools

import jax
import jax.numpy as jnp
from jax import lax
from jax.experimental import pallas as pl
from jax.experimental.pallas import tpu as pltpu
from jax.experimental.pallas import tpu_sc as plsc

B = 4096
S = 200
D = 32
NW = 32
NB = B // NW
R = 8
TR = D // 8
TILE = 8 * NB

_mesh = plsc.VectorSubcoreMesh(core_axis_name="c", subcore_axis_name="s")


@functools.partial(
    pl.kernel,
    mesh=_mesh,
    compiler_params=pltpu.CompilerParams(
        use_tc_tiling_on_sc=False, needs_layout_passes=False),
    out_type=jax.ShapeDtypeStruct((S, TR, NW, 8, NB), jnp.float32),
    scratch_types=[
        pltpu.VMEM((S, NB), jnp.int32),
        pltpu.VMEM((R, NB, D), jnp.float32),
        pltpu.VMEM((D, 136), jnp.float32),
        pltpu.VMEM((D, 136), jnp.float32),
        pltpu.VMEM((S, D), jnp.float32),
        pltpu.SemaphoreType.DMA,
        pltpu.SemaphoreType.DMA,
        pltpu.SemaphoreType.DMA,
        pltpu.SemaphoreType.DMA,
        pltpu.SemaphoreType.DMA,
        pltpu.SemaphoreType.DMA,
        pltpu.SemaphoreType.DMA,
        pltpu.SemaphoreType.DMA,
        pltpu.SemaphoreType.DMA,
        pltpu.SemaphoreType.DMA,
    ],
)
def _emb_kernel(xT_hbm, table_hbm, pe_hbm, out_hbm, idx_v, rows_v, t0_v, t1_v,
                pe_v, sem_g0, sem_g1, sem_g2, sem_g3, sem_g4, sem_g5, sem_g6,
                sem_g7, sem_w0, sem_w1):
    cid = lax.axis_index("c")
    sid = lax.axis_index("s")
    wid = sid * 2 + cid
    sem_g = (sem_g0, sem_g1, sem_g2, sem_g3, sem_g4, sem_g5, sem_g6,
             sem_g7)
    sem_w = (sem_w0, sem_w1)
    t_refs = (t0_v, t1_v)

    pltpu.sync_copy(xT_hbm.at[pl.ds(0, S), pl.ds(NB * wid, NB)], idx_v)
    pltpu.sync_copy(pe_hbm, pe_v)

    iota = lax.iota(jnp.int32, 16)

    def gather_start(s, slot):
        pltpu.async_copy(table_hbm.at[idx_v.at[s]], rows_v.at[slot],
                         sem_g[slot])

    def gather_wait(slot):
        pltpu.make_async_copy(table_hbm.at[idx_v.at[0]], rows_v.at[slot],
                              sem_g[slot]).wait()

    def write_start(s, p):
        for tr in range(TR):
            pltpu.async_copy(t_refs[p].at[pl.ds(8 * tr, 8), pl.ds(0, NB)],
                             out_hbm.at[s].at[tr].at[wid], sem_w[p])

    def write_wait(p):
        for _ in range(TR):
            pltpu.make_async_copy(t_refs[0].at[pl.ds(0, 8), pl.ds(0, NB)],
                                  out_hbm.at[0].at[0].at[0], sem_w[p]).wait()

    for slot in range(R):
        gather_start(slot, slot)

    def s_body(s0, carry):
        for r in range(R):
            s = s0 * R + r
            p = r & 1
            gather_wait(r)
            if r >= 2:
                write_wait(p)
            else:
                @pl.when(s0 >= 1)
                def _():
                    write_wait(p)
            pe_lo = pe_v[s, pl.ds(0, 16)]
            pe_hi = pe_v[s, pl.ds(16, 16)]

            def b_body(j, c2, _r=r, _t=t_refs[p], _pe_lo=pe_lo, _pe_hi=pe_hi):
                for u in range(4):
                    b = j * 4 + u
                    bv = iota * 0 + b
                    lo = rows_v[_r, b, pl.ds(0, 16)] + _pe_lo
                    hi = rows_v[_r, b, pl.ds(16, 16)] + _pe_hi
                    plsc.store_scatter(_t, [iota, bv], lo)
                    plsc.store_scatter(_t, [iota + 16, bv], hi)
                return c2

            lax.fori_loop(0, NB // 4, b_body, 0)

            @pl.when(s0 < S // R - 1)
            def _():
                gather_start(s + R, r)

            write_start(s, p)
        return carry

    lax.fori_loop(0, S // R, s_body, 0)
    write_wait(0)
    write_wait(1)


def kernel(x, table, pe):
    xT = jnp.swapaxes(x, 0, 1).astype(jnp.int32)
    pe2 = pe[0, :S, :]
    out5 = _emb_kernel(xT, table, pe2)
    return out5.transpose(2, 4, 0, 1, 3).reshape(B, S, D)


# speedup vs baseline: 1.0128x; 1.0128x over previous
import functools

import jax
import jax.numpy as jnp
from jax import lax
from jax.experimental import pallas as pl
from jax.experimental.pallas import tpu as pltpu
from jax.experimental.pallas import tpu_sc as plsc

B = 4096
S = 200
D = 32
NW = 32
NB = B // NW
R = 4
TR = D // 8
TILE = 8 * NB

_mesh = plsc.VectorSubcoreMesh(core_axis_name="c", subcore_axis_name="s")


@functools.partial(
    pl.kernel,
    mesh=_mesh,
    compiler_params=pltpu.CompilerParams(
        use_tc_tiling_on_sc=False, needs_layout_passes=False),
    out_type=jax.ShapeDtypeStruct((S, TR, NW, 8, NB), jnp.float32),
    scratch_types=[
        pltpu.VMEM((S, NB), jnp.int32),
        pltpu.VMEM((R, NB, D), jnp.float32),
        pltpu.VMEM((D, 129), jnp.float32),
        pltpu.VMEM((D, 129), jnp.float32),
        pltpu.VMEM((S, D), jnp.float32),
        pltpu.SemaphoreType.DMA,
        pltpu.SemaphoreType.DMA,
        pltpu.SemaphoreType.DMA,
        pltpu.SemaphoreType.DMA,
        pltpu.SemaphoreType.DMA,
        pltpu.SemaphoreType.DMA,
    ],
)
def _emb_kernel(xT_hbm, table_hbm, pe_hbm, out_hbm, idx_v, rows_v, t0_v, t1_v,
                pe_v, sem_g0, sem_g1, sem_g2, sem_g3, sem_w0, sem_w1):
    cid = lax.axis_index("c")
    sid = lax.axis_index("s")
    wid = sid * 2 + cid
    sem_g = (sem_g0, sem_g1, sem_g2, sem_g3)
    sem_w = (sem_w0, sem_w1)
    t_refs = (t0_v, t1_v)

    pltpu.sync_copy(xT_hbm.at[pl.ds(0, S), pl.ds(NB * wid, NB)], idx_v)
    pltpu.sync_copy(pe_hbm, pe_v)

    iota = lax.iota(jnp.int32, 16)

    def gather_start(s, slot):
        pltpu.async_copy(table_hbm.at[idx_v.at[s]], rows_v.at[slot],
                         sem_g[slot])

    def gather_wait(slot):
        pltpu.make_async_copy(table_hbm.at[idx_v.at[0]], rows_v.at[slot],
                              sem_g[slot]).wait()

    def write_start(s, p):
        for tr in range(TR):
            pltpu.async_copy(t_refs[p].at[pl.ds(8 * tr, 8), pl.ds(0, NB)],
                             out_hbm.at[s].at[tr].at[wid], sem_w[p])

    def write_wait(p):
        for _ in range(TR):
            pltpu.make_async_copy(t_refs[0].at[pl.ds(0, 8), pl.ds(0, NB)],
                                  out_hbm.at[0].at[0].at[0], sem_w[p]).wait()

    for slot in range(R):
        gather_start(slot, slot)

    def s_body(s0, carry):
        for r in range(R):
            s = s0 * R + r
            p = r & 1
            gather_wait(r)
            if r >= 2:
                write_wait(p)
            else:
                @pl.when(s0 >= 1)
                def _():
                    write_wait(p)
            pe_lo = pe_v[s, pl.ds(0, 16)]
            pe_hi = pe_v[s, pl.ds(16, 16)]

            def b_body(j, c2, _r=r, _t=t_refs[p], _pe_lo=pe_lo, _pe_hi=pe_hi):
                for u in range(8):
                    b = j * 8 + u
                    bv = iota * 0 + b
                    lo = rows_v[_r, b, pl.ds(0, 16)] + _pe_lo
                    hi = rows_v[_r, b, pl.ds(16, 16)] + _pe_hi
                    plsc.store_scatter(_t, [iota, bv], lo)
                    plsc.store_scatter(_t, [iota + 16, bv], hi)
                return c2

            lax.fori_loop(0, NB // 8, b_body, 0)

            @pl.when(s0 < S // R - 1)
            def _():
                gather_start(s + R, r)

            write_start(s, p)
        return carry

    lax.fori_loop(0, S // R, s_body, 0)
    write_wait(0)
    write_wait(1)


def kernel(x, table, pe):
    xT = jnp.swapaxes(x, 0, 1).astype(jnp.int32)
    pe2 = pe[0, :S, :]
    out5 = _emb_kernel(xT, table, pe2)
    return out5.transpose(2, 4, 0, 1, 3).reshape(B, S, D)


# speedup vs baseline: 1.0212x; 1.0083x over previous
import functools

import jax
import jax.numpy as jnp
from jax import lax
from jax.experimental import pallas as pl
from jax.experimental.pallas import tpu as pltpu
from jax.experimental.pallas import tpu_sc as plsc

B = 4096
S = 200
D = 32
NW = 32
NB = B // NW
R = 4
TR = D // 8
TILE = 8 * NB

_mesh = plsc.VectorSubcoreMesh(core_axis_name="c", subcore_axis_name="s")


@functools.partial(
    pl.kernel,
    mesh=_mesh,
    compiler_params=pltpu.CompilerParams(
        use_tc_tiling_on_sc=False, needs_layout_passes=False),
    out_type=jax.ShapeDtypeStruct((S, TR, NW, 8, NB), jnp.float32),
    scratch_types=[
        pltpu.VMEM((S, NB), jnp.int32),
        pltpu.VMEM((R, NB, D), jnp.float32),
        pltpu.VMEM((TR, 8, 129), jnp.float32),
        pltpu.VMEM((TR, 8, 129), jnp.float32),
        pltpu.VMEM((S, D), jnp.float32),
        pltpu.SemaphoreType.DMA,
        pltpu.SemaphoreType.DMA,
        pltpu.SemaphoreType.DMA,
        pltpu.SemaphoreType.DMA,
        pltpu.SemaphoreType.DMA,
        pltpu.SemaphoreType.DMA,
    ],
)
def _emb_kernel(xT_hbm, table_hbm, pe_hbm, out_hbm, idx_v, rows_v, t0_v, t1_v,
                pe_v, sem_g0, sem_g1, sem_g2, sem_g3, sem_w0, sem_w1):
    cid = lax.axis_index("c")
    sid = lax.axis_index("s")
    wid = sid * 2 + cid
    sem_g = (sem_g0, sem_g1, sem_g2, sem_g3)
    sem_w = (sem_w0, sem_w1)
    t_refs = (t0_v, t1_v)

    pltpu.sync_copy(xT_hbm.at[pl.ds(0, S), pl.ds(NB * wid, NB)], idx_v)
    pltpu.sync_copy(pe_hbm, pe_v)

    iota = lax.iota(jnp.int32, 16)
    dd_c = iota & 7
    tr_lo = iota >> 3
    tr_hi = tr_lo + 2

    def gather_start(s, slot):
        pltpu.async_copy(table_hbm.at[idx_v.at[s]], rows_v.at[slot],
                         sem_g[slot])

    def gather_wait(slot):
        pltpu.make_async_copy(table_hbm.at[idx_v.at[0]], rows_v.at[slot],
                              sem_g[slot]).wait()

    def write_start(s, p):
        pltpu.async_copy(
            t_refs[p].at[pl.ds(0, TR), pl.ds(0, 8), pl.ds(0, NB)],
            out_hbm.at[s, pl.ds(0, TR), wid], sem_w[p])

    def write_wait(p):
        pltpu.make_async_copy(
            t_refs[0].at[pl.ds(0, TR), pl.ds(0, 8), pl.ds(0, NB)],
            out_hbm.at[0, pl.ds(0, TR), 0], sem_w[p]).wait()

    for slot in range(R):
        gather_start(slot, slot)

    def s_body(s0, carry):
        for r in range(R):
            s = s0 * R + r
            p = r & 1
            gather_wait(r)
            if r >= 2:
                write_wait(p)
            else:
                @pl.when(s0 >= 1)
                def _():
                    write_wait(p)
            pe_lo = pe_v[s, pl.ds(0, 16)]
            pe_hi = pe_v[s, pl.ds(16, 16)]

            def b_body(j, c2, _r=r, _t=t_refs[p], _pe_lo=pe_lo, _pe_hi=pe_hi):
                for u in range(8):
                    b = j * 8 + u
                    bv = iota * 0 + b
                    lo = rows_v[_r, b, pl.ds(0, 16)] + _pe_lo
                    hi = rows_v[_r, b, pl.ds(16, 16)] + _pe_hi
                    plsc.store_scatter(_t, [tr_lo, dd_c, bv], lo)
                    plsc.store_scatter(_t, [tr_hi, dd_c, bv], hi)
                return c2

            lax.fori_loop(0, NB // 8, b_body, 0)

            @pl.when(s0 < S // R - 1)
            def _():
                gather_start(s + R, r)

            write_start(s, p)
        return carry

    lax.fori_loop(0, S // R, s_body, 0)
    write_wait(0)
    write_wait(1)


def kernel(x, table, pe):
    xT = jnp.swapaxes(x, 0, 1).astype(jnp.int32)
    pe2 = pe[0, :S, :]
    out5 = _emb_kernel(xT, table, pe2)
    return out5.transpose(2, 4, 0, 1, 3).reshape(B, S, D)
